# baseline (speedup 1.0000x reference)
import functools

import jax
import jax.numpy as jnp
from jax import lax
from jax.experimental import pallas as pl
from jax.experimental.pallas import tpu as pltpu
from jax.experimental.pallas import tpu_sc as plsc

NUM_TOKENS = 16384
D_HIDDEN = 4096
NUM_EXPERTS = 64
TOP_K = 8
BLK = 512
NUM_CHUNKS = 4

_NC = 2
_NS = 16
_NW = _NC * _NS


def _dense_body(x_ref, w_ref, scores_ref):
    x = x_ref[...]
    w = w_ref[...]
    logits = lax.dot_general(
        x, w, (((1,), (1,)), ((), ())), preferred_element_type=jnp.float32
    )
    m = jnp.max(logits, axis=1, keepdims=True)
    e = jnp.exp(logits - m)
    s = jnp.sum(e, axis=1, keepdims=True)
    scores_ref[...] = e / s


def _make_dense(nt):
    return pl.pallas_call(
        _dense_body,
        grid=(nt // BLK,),
        in_specs=[
            pl.BlockSpec((BLK, D_HIDDEN), lambda i: (i, 0)),
            pl.BlockSpec((NUM_EXPERTS, D_HIDDEN), lambda i: (0, 0)),
        ],
        out_specs=pl.BlockSpec((BLK, NUM_EXPERTS), lambda i: (i, 0)),
        out_shape=jax.ShapeDtypeStruct((nt, NUM_EXPERTS), jnp.float32),
    )


def _make_sc_topk(nt):
    tpw = nt // _NW

    def body(scores_hbm, idx_hbm, tks_hbm, scores_v, idx_v, tks_v):
        wid = lax.axis_index("s") * _NC + lax.axis_index("c")
        pltpu.sync_copy(
            scores_hbm.at[pl.ds(wid * (tpw * NUM_EXPERTS), tpw * NUM_EXPERTS)],
            scores_v,
        )

        iota = lax.iota(jnp.int32, 16)
        lm = iota < 8

        @plsc.parallel_loop(0, tpw, unroll=8)
        def token_body(t):
            base = t * NUM_EXPERTS
            s0 = scores_v[pl.ds(base, 16)]
            s1 = scores_v[pl.ds(base + 16, 16)]
            s2 = scores_v[pl.ds(base + 32, 16)]
            s3 = scores_v[pl.ds(base + 48, 16)]
            k0, v0 = plsc.sort_key_val(s0, iota, descending=True)
            k1, v1 = plsc.sort_key_val(s1, iota + 16, descending=False)
            k2, v2 = plsc.sort_key_val(s2, iota + 32, descending=True)
            k3, v3 = plsc.sort_key_val(s3, iota + 48, descending=False)
            c1k = jnp.where(lm, k0, k1)
            c1v = jnp.where(lm, v0, v1)
            c2k = jnp.where(lm, k2, k3)
            c2v = jnp.where(lm, v2, v3)
            d1k, d1v = plsc.sort_key_val(c1k, c1v, descending=True)
            d2k, d2v = plsc.sort_key_val(c2k, c2v, descending=False)
            fk0 = jnp.where(lm, d1k, d2k)
            fv0 = jnp.where(lm, d1v, d2v)
            fk, fv = plsc.sort_key_val(fk0, fv0, descending=True)
            ssum = jnp.sum(jnp.where(lm, fk, 0.0), axis=0)
            tks = fk / ssum
            plsc.store_scatter(idx_v, [t * TOP_K + iota], fv, mask=lm)
            plsc.store_scatter(tks_v, [t * TOP_K + iota], tks, mask=lm)

        pltpu.sync_copy(idx_v, idx_hbm.at[pl.ds(wid * (tpw * TOP_K), tpw * TOP_K)])
        pltpu.sync_copy(tks_v, tks_hbm.at[pl.ds(wid * (tpw * TOP_K), tpw * TOP_K)])

    return pl.kernel(
        body,
        mesh=plsc.VectorSubcoreMesh(core_axis_name="c", subcore_axis_name="s"),
        out_type=(
            jax.ShapeDtypeStruct((nt * TOP_K,), jnp.int32),
            jax.ShapeDtypeStruct((nt * TOP_K,), jnp.float32),
        ),
        scratch_types=[
            pltpu.VMEM((tpw * NUM_EXPERTS,), jnp.float32),
            pltpu.VMEM((tpw * TOP_K,), jnp.int32),
            pltpu.VMEM((tpw * TOP_K,), jnp.float32),
        ],
        compiler_params=pltpu.CompilerParams(needs_layout_passes=False),
    )


_CT = NUM_TOKENS // NUM_CHUNKS
_dense_chunk = _make_dense(_CT)
_sc_topk_chunk = _make_sc_topk(_CT)


def kernel(x, W_g):
    scores_parts, idx_parts, tks_parts = [], [], []
    for c in range(NUM_CHUNKS):
        xc = lax.slice(x, (c * _CT, 0), ((c + 1) * _CT, D_HIDDEN))
        sc = _dense_chunk(xc, W_g)
        idx_c, tks_c = _sc_topk_chunk(sc.reshape(-1))
        scores_parts.append(sc)
        idx_parts.append(idx_c.reshape(_CT, TOP_K))
        tks_parts.append(tks_c.reshape(_CT, TOP_K))
    return (
        jnp.concatenate(idx_parts, axis=0),
        jnp.concatenate(tks_parts, axis=0),
        jnp.concatenate(scores_parts, axis=0),
    )


# speedup vs baseline: 2.0355x; 2.0355x over previous
import functools

import jax
import jax.numpy as jnp
from jax import lax
from jax.experimental import pallas as pl
from jax.experimental.pallas import tpu as pltpu
from jax.experimental.pallas import tpu_sc as plsc

NUM_TOKENS = 16384
D_HIDDEN = 4096
NUM_EXPERTS = 64
TOP_K = 8
BLK = 512
NUM_CHUNKS = 4

_NC = 2
_NS = 16
_NW = _NC * _NS


def _dense_body(x_ref, w_ref, scores_ref):
    x = x_ref[...]
    w = w_ref[...]
    logits = lax.dot_general(
        x, w, (((1,), (1,)), ((), ())), preferred_element_type=jnp.float32
    )
    m = jnp.max(logits, axis=1, keepdims=True)
    e = jnp.exp(logits - m)
    s = jnp.sum(e, axis=1, keepdims=True)
    scores_ref[...] = e / s


def _make_dense(nt, chunk):
    off = chunk * (nt // BLK)
    return pl.pallas_call(
        _dense_body,
        grid=(nt // BLK,),
        in_specs=[
            pl.BlockSpec((BLK, D_HIDDEN), lambda i: (off + i, 0)),
            pl.BlockSpec((NUM_EXPERTS, D_HIDDEN), lambda i: (0, 0)),
        ],
        out_specs=pl.BlockSpec((BLK, NUM_EXPERTS), lambda i: (i, 0)),
        out_shape=jax.ShapeDtypeStruct((nt, NUM_EXPERTS), jnp.float32),
    )


def _make_sc_topk(nt):
    tpw = nt // _NW

    def body(scores_hbm, idx_hbm, tks_hbm, scores_v, idx_v, tks_v):
        wid = lax.axis_index("s") * _NC + lax.axis_index("c")
        pltpu.sync_copy(
            scores_hbm.at[pl.ds(wid * (tpw * NUM_EXPERTS), tpw * NUM_EXPERTS)],
            scores_v,
        )

        iota = lax.iota(jnp.int32, 16)
        lm = iota < 8

        @plsc.parallel_loop(0, tpw, unroll=8)
        def token_body(t):
            base = t * NUM_EXPERTS
            s0 = scores_v[pl.ds(base, 16)]
            s1 = scores_v[pl.ds(base + 16, 16)]
            s2 = scores_v[pl.ds(base + 32, 16)]
            s3 = scores_v[pl.ds(base + 48, 16)]
            k0, v0 = plsc.sort_key_val(s0, iota, descending=True)
            k1, v1 = plsc.sort_key_val(s1, iota + 16, descending=False)
            k2, v2 = plsc.sort_key_val(s2, iota + 32, descending=True)
            k3, v3 = plsc.sort_key_val(s3, iota + 48, descending=False)
            c1k = jnp.where(lm, k0, k1)
            c1v = jnp.where(lm, v0, v1)
            c2k = jnp.where(lm, k2, k3)
            c2v = jnp.where(lm, v2, v3)
            d1k, d1v = plsc.sort_key_val(c1k, c1v, descending=True)
            d2k, d2v = plsc.sort_key_val(c2k, c2v, descending=False)
            fk0 = jnp.where(lm, d1k, d2k)
            fv0 = jnp.where(lm, d1v, d2v)
            fk, fv = plsc.sort_key_val(fk0, fv0, descending=True)
            ssum = jnp.sum(jnp.where(lm, fk, 0.0), axis=0)
            tks = fk / ssum
            plsc.store_scatter(idx_v, [t * TOP_K + iota], fv, mask=lm)
            plsc.store_scatter(tks_v, [t * TOP_K + iota], tks, mask=lm)

        pltpu.sync_copy(idx_v, idx_hbm.at[pl.ds(wid * (tpw * TOP_K), tpw * TOP_K)])
        pltpu.sync_copy(tks_v, tks_hbm.at[pl.ds(wid * (tpw * TOP_K), tpw * TOP_K)])

    return pl.kernel(
        body,
        mesh=plsc.VectorSubcoreMesh(core_axis_name="c", subcore_axis_name="s"),
        out_type=(
            jax.ShapeDtypeStruct((nt * TOP_K,), jnp.int32),
            jax.ShapeDtypeStruct((nt * TOP_K,), jnp.float32),
        ),
        scratch_types=[
            pltpu.VMEM((tpw * NUM_EXPERTS,), jnp.float32),
            pltpu.VMEM((tpw * TOP_K,), jnp.int32),
            pltpu.VMEM((tpw * TOP_K,), jnp.float32),
        ],
        compiler_params=pltpu.CompilerParams(needs_layout_passes=False),
    )


_CT = NUM_TOKENS // NUM_CHUNKS
_dense_chunks = [_make_dense(_CT, c) for c in range(NUM_CHUNKS)]
_sc_topk_chunk = _make_sc_topk(_CT)


def kernel(x, W_g):
    scores_parts, idx_parts, tks_parts = [], [], []
    for c in range(NUM_CHUNKS):
        sc = _dense_chunks[c](x, W_g)
        idx_c, tks_c = _sc_topk_chunk(sc.reshape(-1))
        scores_parts.append(sc)
        idx_parts.append(idx_c.reshape(_CT, TOP_K))
        tks_parts.append(tks_c.reshape(_CT, TOP_K))
    return (
        jnp.concatenate(idx_parts, axis=0),
        jnp.concatenate(tks_parts, axis=0),
        jnp.concatenate(scores_parts, axis=0),
    )


# speedup vs baseline: 2.2729x; 1.1166x over previous
import functools

import jax
import jax.numpy as jnp
from jax import lax
from jax.experimental import pallas as pl
from jax.experimental.pallas import tpu as pltpu
from jax.experimental.pallas import tpu_sc as plsc

NUM_TOKENS = 16384
D_HIDDEN = 4096
NUM_EXPERTS = 64
TOP_K = 8
BLK = 512
NUM_CHUNKS = 4

_NC = 2
_NS = 16
_NW = _NC * _NS


def _dense_body(x_ref, w_ref, scores_ref):
    x = x_ref[...]
    w = w_ref[...]
    logits = lax.dot_general(
        x, w, (((1,), (1,)), ((), ())), preferred_element_type=jnp.float32
    )
    m = jnp.max(logits, axis=1, keepdims=True)
    e = jnp.exp(logits - m)
    s = jnp.sum(e, axis=1, keepdims=True)
    scores_ref[...] = e / s


def _make_dense(nt, chunk):
    off = chunk * (nt // BLK)
    return pl.pallas_call(
        _dense_body,
        grid=(nt // BLK,),
        in_specs=[
            pl.BlockSpec((BLK, D_HIDDEN), lambda i: (off + i, 0)),
            pl.BlockSpec((NUM_EXPERTS, D_HIDDEN), lambda i: (0, 0)),
        ],
        out_specs=pl.BlockSpec((BLK, NUM_EXPERTS), lambda i: (i, 0)),
        out_shape=jax.ShapeDtypeStruct((nt, NUM_EXPERTS), jnp.float32),
    )


def _make_sc_topk(nt):
    tpw = nt // _NW

    def body(scores_hbm, idx_hbm, tks_hbm, scores_v, idx_v, tks_v):
        wid = lax.axis_index("s") * _NC + lax.axis_index("c")
        pltpu.sync_copy(
            scores_hbm.at[pl.ds(wid * (tpw * NUM_EXPERTS), tpw * NUM_EXPERTS)],
            scores_v,
        )

        iota = lax.iota(jnp.int32, 16)
        lm = iota < 8

        @plsc.parallel_loop(0, tpw, unroll=8)
        def token_body(t):
            base = t * NUM_EXPERTS
            s0 = scores_v[pl.ds(base, 16)]
            s1 = scores_v[pl.ds(base + 16, 16)]
            s2 = scores_v[pl.ds(base + 32, 16)]
            s3 = scores_v[pl.ds(base + 48, 16)]
            k0, v0 = plsc.sort_key_val(s0, iota, descending=True)
            k1, v1 = plsc.sort_key_val(s1, iota + 16, descending=False)
            k2, v2 = plsc.sort_key_val(s2, iota + 32, descending=True)
            k3, v3 = plsc.sort_key_val(s3, iota + 48, descending=False)
            c1k = jnp.where(lm, k0, k1)
            c1v = jnp.where(lm, v0, v1)
            c2k = jnp.where(lm, k2, k3)
            c2v = jnp.where(lm, v2, v3)
            d1k, d1v = plsc.sort_key_val(c1k, c1v, descending=True)
            d2k, d2v = plsc.sort_key_val(c2k, c2v, descending=False)
            fk0 = jnp.where(lm, d1k, d2k)
            fv0 = jnp.where(lm, d1v, d2v)
            fk, fv = plsc.sort_key_val(fk0, fv0, descending=True)
            ssum = jnp.sum(jnp.where(lm, fk, 0.0), axis=0)
            tks = fk / ssum
            plsc.store_scatter(idx_v, [t * TOP_K + iota], fv, mask=lm)
            plsc.store_scatter(tks_v, [t * TOP_K + iota], tks, mask=lm)

        pltpu.sync_copy(idx_v, idx_hbm.at[pl.ds(wid * (tpw * TOP_K), tpw * TOP_K)])
        pltpu.sync_copy(tks_v, tks_hbm.at[pl.ds(wid * (tpw * TOP_K), tpw * TOP_K)])

    return pl.kernel(
        body,
        mesh=plsc.VectorSubcoreMesh(core_axis_name="c", subcore_axis_name="s"),
        out_type=(
            jax.ShapeDtypeStruct((nt * TOP_K,), jnp.int32),
            jax.ShapeDtypeStruct((nt * TOP_K,), jnp.float32),
        ),
        scratch_types=[
            pltpu.VMEM((tpw * NUM_EXPERTS,), jnp.float32),
            pltpu.VMEM((tpw * TOP_K,), jnp.int32),
            pltpu.VMEM((tpw * TOP_K,), jnp.float32),
        ],
        compiler_params=pltpu.CompilerParams(needs_layout_passes=False),
    )


_dense_full = _make_dense(NUM_TOKENS, 0)
_sc_topk_full = _make_sc_topk(NUM_TOKENS)


def kernel(x, W_g):
    scores = _dense_full(x, W_g)
    idx_flat, tks_flat = _sc_topk_full(scores.reshape(-1))
    return (
        idx_flat.reshape(NUM_TOKENS, TOP_K),
        tks_flat.reshape(NUM_TOKENS, TOP_K),
        scores,
    )
